# baseline (speedup 1.0000x reference)
import jax
import jax.numpy as jnp
from jax import lax
from jax.experimental import pallas as pl
from jax.experimental.pallas import tpu as pltpu
from jax.experimental.pallas import tpu_sc as plsc

VOCAB = 1000000
EMB = 64
B = 4096
L = 200

NC = 2
NS = 16
NW = NC * NS

PER_W = B // NW
SPLIT = 128
NBUF = 4
GLAG = 2
OUTER = PER_W // NBUF


def _body(idx_hbm, table_hbm, out_hbm, idx_v, rows_v, gsem, ssem):
    wid = lax.axis_index("s") * NC + lax.axis_index("c")
    base = wid * PER_W
    pltpu.sync_copy(idx_hbm.at[pl.ds(base, PER_W)], idx_v)

    def fire_gather(r, b):
        pltpu.async_copy(
            table_hbm.at[idx_v.at[r, pl.ds(0, SPLIT)]],
            rows_v.at[b, pl.ds(0, SPLIT)],
            gsem.at[b],
        )
        pltpu.async_copy(
            table_hbm.at[idx_v.at[r, pl.ds(SPLIT, L - SPLIT)]],
            rows_v.at[b, pl.ds(SPLIT, L - SPLIT)],
            gsem.at[b],
        )

    def wait_gather(r, b):
        pltpu.make_async_copy(
            table_hbm.at[idx_v.at[r]], rows_v.at[b], gsem.at[b]
        ).wait()

    def fire_store(r, b):
        pltpu.async_copy(rows_v.at[b], out_hbm.at[base + r], ssem.at[b])

    def wait_store(r, b):
        pltpu.make_async_copy(
            rows_v.at[b], out_hbm.at[base + r], ssem.at[b]
        ).wait()

    for b in range(NBUF):
        fire_gather(b, b)
    for j in range(NBUF - GLAG):
        wait_gather(j, j)
        fire_store(j, j)

    def outer(o, carry):
        for b in range(NBUF):
            i = o * NBUF + b
            wait_store(i - NBUF, b)
            fire_gather(i, b)
            j = i - GLAG
            bj = (b + NBUF - GLAG) % NBUF
            wait_gather(j, bj)
            fire_store(j, bj)
        return carry

    lax.fori_loop(1, OUTER, outer, 0)

    last = OUTER * NBUF
    for k in range(GLAG):
        i = last - GLAG + k
        b = i % NBUF
        wait_gather(i, b)
        fire_store(i, b)
    for k in range(NBUF):
        i = last - NBUF + k
        wait_store(i, i % NBUF)


@jax.jit
def kernel(idx, table):
    mesh = plsc.VectorSubcoreMesh(
        core_axis_name="c", subcore_axis_name="s", num_cores=NC, num_subcores=NS
    )
    out = pl.kernel(
        _body,
        out_type=jax.ShapeDtypeStruct((B, L, EMB), jnp.float32),
        mesh=mesh,
        scratch_types=[
            pltpu.VMEM((PER_W, L), jnp.int32),
            pltpu.VMEM((NBUF, L, EMB), jnp.float32),
            pltpu.SemaphoreType.DMA((NBUF,)),
            pltpu.SemaphoreType.DMA((NBUF,)),
        ],
        compiler_params=pltpu.CompilerParams(use_tc_tiling_on_sc=False),
    )(idx.astype(jnp.int32), table)
    return out
